# baseline (speedup 1.0000x reference)
import dataclasses
import functools

import jax
import jax.numpy as jnp
from jax import lax
from jax.experimental import pallas as pl
from jax.experimental.pallas import tpu as pltpu
from jax.experimental.pallas import tpu_sc as plsc

N = 524288
K = 512
NC = 2
NS = 16
NW = NC * NS
L = 16
CHUNK = N // NW


def _prep_body(cb_col_ref, cb_row_ref, sorted_ref, mids_ref):
    ci = cb_col_ref[...]
    cj = cb_row_ref[...]
    ii = lax.broadcasted_iota(jnp.int32, (K, K), 0)
    jj = lax.broadcasted_iota(jnp.int32, (K, K), 1)
    gt = (ci > cj) | ((ci == cj) & (ii > jj))
    rank = jnp.sum(gt.astype(jnp.int32), axis=1, keepdims=True)
    m1 = rank == jj
    m2 = rank == jj + 1
    sorted_row = jnp.sum(jnp.where(m1, ci, 0.0), axis=0, keepdims=True)
    pairsum = jnp.sum(jnp.where(m1 | m2, ci, 0.0), axis=0, keepdims=True)
    k_row = lax.broadcasted_iota(jnp.int32, (1, K), 1)
    mids_row = jnp.where(k_row == K - 1, jnp.inf, 0.5 * pairsum)
    sorted_ref[...] = sorted_row
    mids_ref[...] = mids_row


_prep = pl.pallas_call(
    _prep_body,
    out_shape=(
        jax.ShapeDtypeStruct((1, K), jnp.float32),
        jax.ShapeDtypeStruct((1, K), jnp.float32),
    ),
)


def _vq_body(param_hbm, sorted_hbm, mids_hbm, out_hbm, cb_v, mids_v, x_v, o_v):
    wid = lax.axis_index("s") * NC + lax.axis_index("c")
    base = wid * CHUNK
    pltpu.sync_copy(sorted_hbm, cb_v)
    pltpu.sync_copy(mids_hbm, mids_v)
    pltpu.sync_copy(param_hbm.at[pl.ds(base, CHUNK)], x_v)

    @plsc.parallel_loop(0, CHUNK, step=L, unroll=16)
    def _(i):
        x = x_v[pl.ds(i, L)]
        b = jnp.zeros((L,), jnp.int32)
        for half in (256, 128, 64, 32, 16, 8, 4, 2, 1):
            m = plsc.load_gather(mids_v, [b + (half - 1)])
            b = jnp.where(m <= x, b + half, b)
        o_v[pl.ds(i, L)] = plsc.load_gather(cb_v, [b])

    pltpu.sync_copy(o_v, out_hbm.at[pl.ds(base, CHUNK)])


@functools.cache
def _make_vq():
    cp = pltpu.CompilerParams()
    if "needs_layout_passes" in pltpu.CompilerParams.__dataclass_fields__:
        cp = dataclasses.replace(cp, needs_layout_passes=False)
    return pl.kernel(
        _vq_body,
        compiler_params=cp,
        out_type=jax.ShapeDtypeStruct((N,), jnp.float32),
        mesh=plsc.VectorSubcoreMesh(core_axis_name="c", subcore_axis_name="s"),
        scratch_types=[
            pltpu.VMEM((K,), jnp.float32),
            pltpu.VMEM((K,), jnp.float32),
            pltpu.VMEM((CHUNK,), jnp.float32),
            pltpu.VMEM((CHUNK,), jnp.float32),
        ],
    )


@jax.jit
def kernel(param, y, codebook):
    cb_col = codebook.reshape(K, 1)
    cb_row = codebook.reshape(1, K)
    sorted_cb, mids = _prep(cb_col, cb_row)
    quantized = _make_vq()(param, sorted_cb.reshape(K), mids.reshape(K))
    return (quantized, y)


# speedup vs baseline: 1.0155x; 1.0155x over previous
import dataclasses
import functools

import jax
import jax.numpy as jnp
from jax import lax
from jax.experimental import pallas as pl
from jax.experimental.pallas import tpu as pltpu
from jax.experimental.pallas import tpu_sc as plsc

N = 524288
K = 512
NC = 2
NS = 16
NW = NC * NS
L = 16
CHUNK = N // NW


def _prep_body(cb_col_ref, cb_row_ref, sorted_ref, mids_ref):
    ci = cb_col_ref[...]
    cj = cb_row_ref[...]
    ii = lax.broadcasted_iota(jnp.int32, (K, K), 0)
    jj = lax.broadcasted_iota(jnp.int32, (K, K), 1)
    gt = (ci > cj) | ((ci == cj) & (ii > jj))
    rank = jnp.sum(gt.astype(jnp.int32), axis=1, keepdims=True)
    m1 = rank == jj
    m2 = rank == jj + 1
    sorted_row = jnp.sum(jnp.where(m1, ci, 0.0), axis=0, keepdims=True)
    pairsum = jnp.sum(jnp.where(m1 | m2, ci, 0.0), axis=0, keepdims=True)
    k_row = lax.broadcasted_iota(jnp.int32, (1, K), 1)
    mids_row = jnp.where(k_row == K - 1, jnp.inf, 0.5 * pairsum)
    sorted_ref[...] = sorted_row
    mids_ref[...] = mids_row


_prep = pl.pallas_call(
    _prep_body,
    out_shape=(
        jax.ShapeDtypeStruct((1, K), jnp.float32),
        jax.ShapeDtypeStruct((1, K), jnp.float32),
    ),
)


def _vq_body(param_hbm, sorted_hbm, mids_hbm, out_hbm, cb_v, mids_v, x_v, o_v):
    wid = lax.axis_index("s") * NC + lax.axis_index("c")
    base = wid * CHUNK
    pltpu.sync_copy(sorted_hbm, cb_v)
    pltpu.sync_copy(mids_hbm, mids_v)
    pltpu.sync_copy(param_hbm.at[pl.ds(base, CHUNK)], x_v)

    @plsc.parallel_loop(0, CHUNK, step=L, unroll=8)
    def _(i):
        x = x_v[pl.ds(i, L)]
        b = jnp.zeros((L,), jnp.int32)
        for half in (256, 128, 64, 32, 16, 8, 4, 2, 1):
            m = plsc.load_gather(mids_v, [b + (half - 1)])
            b = jnp.where(m <= x, b + half, b)
        o_v[pl.ds(i, L)] = plsc.load_gather(cb_v, [b])

    pltpu.sync_copy(o_v, out_hbm.at[pl.ds(base, CHUNK)])


@functools.cache
def _make_vq():
    cp = pltpu.CompilerParams()
    if "needs_layout_passes" in pltpu.CompilerParams.__dataclass_fields__:
        cp = dataclasses.replace(cp, needs_layout_passes=False)
    return pl.kernel(
        _vq_body,
        compiler_params=cp,
        out_type=jax.ShapeDtypeStruct((N,), jnp.float32),
        mesh=plsc.VectorSubcoreMesh(core_axis_name="c", subcore_axis_name="s"),
        scratch_types=[
            pltpu.VMEM((K,), jnp.float32),
            pltpu.VMEM((K,), jnp.float32),
            pltpu.VMEM((CHUNK,), jnp.float32),
            pltpu.VMEM((CHUNK,), jnp.float32),
        ],
    )


@jax.jit
def kernel(param, y, codebook):
    cb_col = codebook.reshape(K, 1)
    cb_row = codebook.reshape(1, K)
    sorted_cb, mids = _prep(cb_col, cb_row)
    quantized = _make_vq()(param, sorted_cb.reshape(K), mids.reshape(K))
    return (quantized, y)


# speedup vs baseline: 1.3230x; 1.3028x over previous
import dataclasses
import functools

import jax
import jax.numpy as jnp
from jax import lax
from jax.experimental import pallas as pl
from jax.experimental.pallas import tpu as pltpu
from jax.experimental.pallas import tpu_sc as plsc

N = 524288
K = 512
NC = 2
NS = 16
NW = NC * NS
L = 16
CHUNK = N // NW


def _prep_body(cb_col_ref, cb_row_ref, sorted_ref, mids_ref):
    ci = cb_col_ref[...]
    cj = cb_row_ref[...]
    ii = lax.broadcasted_iota(jnp.int32, (K, K), 0)
    jj = lax.broadcasted_iota(jnp.int32, (K, K), 1)
    gt = (ci > cj) | ((ci == cj) & (ii > jj))
    rank = jnp.sum(gt.astype(jnp.int32), axis=1, keepdims=True)
    m1 = rank == jj
    m2 = rank == jj + 1
    sorted_row = jnp.sum(jnp.where(m1, ci, 0.0), axis=0, keepdims=True)
    pairsum = jnp.sum(jnp.where(m1 | m2, ci, 0.0), axis=0, keepdims=True)
    k_row = lax.broadcasted_iota(jnp.int32, (1, K), 1)
    mids_row = jnp.where(k_row == K - 1, jnp.inf, 0.5 * pairsum)
    sorted_ref[...] = sorted_row
    mids_ref[...] = mids_row


_prep = pl.pallas_call(
    _prep_body,
    out_shape=(
        jax.ShapeDtypeStruct((1, K), jnp.float32),
        jax.ShapeDtypeStruct((1, K), jnp.float32),
    ),
)


def _vq_body(param_hbm, sorted_hbm, mids_hbm, out_hbm, cb_v, mids_v, x_v, o_v):
    wid = lax.axis_index("s") * NC + lax.axis_index("c")
    base = wid * CHUNK
    pltpu.sync_copy(sorted_hbm, cb_v)
    pltpu.sync_copy(mids_hbm, mids_v)
    pltpu.sync_copy(param_hbm.at[pl.ds(base, CHUNK)], x_v)

    stride = K // 16
    sp_idx = lax.iota(jnp.int32, L) * stride + (stride - 1)
    sp_vec = plsc.load_gather(mids_v, [sp_idx])
    splits = [jnp.full((L,), sp_vec[t - 1]) for t in range(1, 16)]

    @plsc.parallel_loop(0, CHUNK, step=L, unroll=8)
    def _(i):
        x = x_v[pl.ds(i, L)]
        b = jnp.zeros((L,), jnp.int32)
        for s in splits:
            b = jnp.where(s <= x, b + stride, b)
        for half in (16, 8, 4, 2, 1):
            m = plsc.load_gather(mids_v, [b + (half - 1)])
            b = jnp.where(m <= x, b + half, b)
        o_v[pl.ds(i, L)] = plsc.load_gather(cb_v, [b])

    pltpu.sync_copy(o_v, out_hbm.at[pl.ds(base, CHUNK)])


@functools.cache
def _make_vq():
    cp = pltpu.CompilerParams()
    if "needs_layout_passes" in pltpu.CompilerParams.__dataclass_fields__:
        cp = dataclasses.replace(cp, needs_layout_passes=False)
    return pl.kernel(
        _vq_body,
        compiler_params=cp,
        out_type=jax.ShapeDtypeStruct((N,), jnp.float32),
        mesh=plsc.VectorSubcoreMesh(core_axis_name="c", subcore_axis_name="s"),
        scratch_types=[
            pltpu.VMEM((K,), jnp.float32),
            pltpu.VMEM((K,), jnp.float32),
            pltpu.VMEM((CHUNK,), jnp.float32),
            pltpu.VMEM((CHUNK,), jnp.float32),
        ],
    )


@jax.jit
def kernel(param, y, codebook):
    cb_col = codebook.reshape(K, 1)
    cb_row = codebook.reshape(1, K)
    sorted_cb, mids = _prep(cb_col, cb_row)
    quantized = _make_vq()(param, sorted_cb.reshape(K), mids.reshape(K))
    return (quantized, y)


# speedup vs baseline: 1.8769x; 1.4187x over previous
import dataclasses
import functools

import numpy as np

import jax
import jax.numpy as jnp
from jax import lax
from jax.experimental import pallas as pl
from jax.experimental.pallas import tpu as pltpu
from jax.experimental.pallas import tpu_sc as plsc

N = 524288
K = 512
NC = 2
NS = 16
NW = NC * NS
L = 16
CHUNK = N // NW
T_LEN = 2 * K
REG_LEVELS = 4
GATHER_LEVELS = 9 - REG_LEVELS


def _tree_index_maps():
    perm = np.zeros(K - 1, np.int32)
    for j in range(K - 1):
        lvl = (j + 1).bit_length() - 1
        pos = j - ((1 << lvl) - 1)
        span = K >> lvl
        perm[j] = pos * span + span // 2 - 1
    a = np.zeros(T_LEN, np.int32)
    b = np.zeros(T_LEN, np.int32)
    a[: K - 1] = perm
    b[: K - 1] = perm + 1
    a[K - 1 : T_LEN - 1] = b[K - 1 : T_LEN - 1] = np.arange(K)
    return a.reshape(1, T_LEN), b.reshape(1, T_LEN)


_A_MAP, _B_MAP = _tree_index_maps()


def _prep_body(cb_col_ref, cb_row_ref, a_ref, b_ref, t_ref):
    ci = cb_col_ref[...]
    cj = cb_row_ref[...]
    ii = lax.broadcasted_iota(jnp.int32, (K, K), 0)
    jj = lax.broadcasted_iota(jnp.int32, (K, K), 1)
    gt = (ci > cj) | ((ci == cj) & (ii > jj))
    rank = jnp.sum(gt.astype(jnp.int32), axis=1, keepdims=True)
    sel = (rank == a_ref[...]).astype(jnp.float32) + (
        rank == b_ref[...]
    ).astype(jnp.float32)
    t_ref[...] = 0.5 * jnp.sum(sel * ci, axis=0, keepdims=True)


_prep = pl.pallas_call(
    _prep_body,
    out_shape=jax.ShapeDtypeStruct((1, T_LEN), jnp.float32),
)


def _vq_body(param_hbm, t_hbm, out_hbm, t_v, x_v, o_v):
    wid = lax.axis_index("s") * NC + lax.axis_index("c")
    base = wid * CHUNK
    pltpu.sync_copy(t_hbm, t_v)
    pltpu.sync_copy(param_hbm.at[pl.ds(base, CHUNK)], x_v)

    sp = t_v[pl.ds(0, L)]
    splits = [jnp.full((L,), sp[t]) for t in range((1 << REG_LEVELS) - 1)]

    @plsc.parallel_loop(0, CHUNK, step=L, unroll=8)
    def _(i):
        x = x_v[pl.ds(i, L)]
        j = jnp.full((L,), (1 << REG_LEVELS) - 1, jnp.int32)
        for s in splits:
            j = jnp.where(s <= x, j + 1, j)
        for _lvl in range(GATHER_LEVELS):
            m = plsc.load_gather(t_v, [j])
            j = j + j + jnp.where(m <= x, 2, 1)
        o_v[pl.ds(i, L)] = plsc.load_gather(t_v, [j])

    pltpu.sync_copy(o_v, out_hbm.at[pl.ds(base, CHUNK)])


@functools.cache
def _make_vq():
    cp = pltpu.CompilerParams()
    if "needs_layout_passes" in pltpu.CompilerParams.__dataclass_fields__:
        cp = dataclasses.replace(cp, needs_layout_passes=False)
    return pl.kernel(
        _vq_body,
        compiler_params=cp,
        out_type=jax.ShapeDtypeStruct((N,), jnp.float32),
        mesh=plsc.VectorSubcoreMesh(core_axis_name="c", subcore_axis_name="s"),
        scratch_types=[
            pltpu.VMEM((T_LEN,), jnp.float32),
            pltpu.VMEM((CHUNK,), jnp.float32),
            pltpu.VMEM((CHUNK,), jnp.float32),
        ],
    )


@jax.jit
def kernel(param, y, codebook):
    cb_col = codebook.reshape(K, 1)
    cb_row = codebook.reshape(1, K)
    tree = _prep(cb_col, cb_row, jnp.asarray(_A_MAP), jnp.asarray(_B_MAP))
    quantized = _make_vq()(param, tree.reshape(T_LEN))
    return (quantized, y)


# speedup vs baseline: 1.8770x; 1.0000x over previous
import dataclasses
import functools

import numpy as np

import jax
import jax.numpy as jnp
from jax import lax
from jax.experimental import pallas as pl
from jax.experimental.pallas import tpu as pltpu
from jax.experimental.pallas import tpu_sc as plsc

N = 524288
K = 512
NC = 2
NS = 16
NW = NC * NS
L = 16
CHUNK = N // NW
T_LEN = 2 * K
REG_LEVELS = 4
GATHER_LEVELS = 9 - REG_LEVELS


def _tree_index_maps():
    perm = np.zeros(K - 1, np.int32)
    for j in range(K - 1):
        lvl = (j + 1).bit_length() - 1
        pos = j - ((1 << lvl) - 1)
        span = K >> lvl
        perm[j] = pos * span + span // 2 - 1
    a = np.zeros(T_LEN, np.int32)
    b = np.zeros(T_LEN, np.int32)
    a[: K - 1] = perm
    b[: K - 1] = perm + 1
    a[K - 1 : T_LEN - 1] = b[K - 1 : T_LEN - 1] = np.arange(K)
    return a.reshape(1, T_LEN), b.reshape(1, T_LEN)


_A_MAP, _B_MAP = _tree_index_maps()


def _prep_body(cb_col_ref, cb_row_ref, a_ref, b_ref, t_ref):
    ci = cb_col_ref[...]
    cj = cb_row_ref[...]
    ii = lax.broadcasted_iota(jnp.int32, (K, K), 0)
    jj = lax.broadcasted_iota(jnp.int32, (K, K), 1)
    gt = (ci > cj) | ((ci == cj) & (ii > jj))
    rank = jnp.sum(gt.astype(jnp.int32), axis=1, keepdims=True)
    sel = (rank == a_ref[...]).astype(jnp.float32) + (
        rank == b_ref[...]
    ).astype(jnp.float32)
    t_ref[...] = 0.5 * jnp.sum(sel * ci, axis=0, keepdims=True)


_prep = pl.pallas_call(
    _prep_body,
    out_shape=jax.ShapeDtypeStruct((1, T_LEN), jnp.float32),
)


def _vq_body(param_hbm, t_hbm, out_hbm, t_v, x_v, o_v):
    wid = lax.axis_index("s") * NC + lax.axis_index("c")
    base = wid * CHUNK
    pltpu.sync_copy(t_hbm, t_v)
    pltpu.sync_copy(param_hbm.at[pl.ds(base, CHUNK)], x_v)

    sp = t_v[pl.ds(0, L)]
    splits = [jnp.full((L,), sp[t]) for t in range((1 << REG_LEVELS) - 1)]

    @plsc.parallel_loop(0, CHUNK, step=L, unroll=4)
    def _(i):
        x = x_v[pl.ds(i, L)]
        j = jnp.full((L,), (1 << REG_LEVELS) - 1, jnp.int32)
        for s in splits:
            j = jnp.where(s <= x, j + 1, j)
        for _lvl in range(GATHER_LEVELS):
            m = plsc.load_gather(t_v, [j])
            j = j + j + jnp.where(m <= x, 2, 1)
        o_v[pl.ds(i, L)] = plsc.load_gather(t_v, [j])

    pltpu.sync_copy(o_v, out_hbm.at[pl.ds(base, CHUNK)])


@functools.cache
def _make_vq():
    cp = pltpu.CompilerParams()
    if "needs_layout_passes" in pltpu.CompilerParams.__dataclass_fields__:
        cp = dataclasses.replace(cp, needs_layout_passes=False)
    return pl.kernel(
        _vq_body,
        compiler_params=cp,
        out_type=jax.ShapeDtypeStruct((N,), jnp.float32),
        mesh=plsc.VectorSubcoreMesh(core_axis_name="c", subcore_axis_name="s"),
        scratch_types=[
            pltpu.VMEM((T_LEN,), jnp.float32),
            pltpu.VMEM((CHUNK,), jnp.float32),
            pltpu.VMEM((CHUNK,), jnp.float32),
        ],
    )


@jax.jit
def kernel(param, y, codebook):
    cb_col = codebook.reshape(K, 1)
    cb_row = codebook.reshape(1, K)
    tree = _prep(cb_col, cb_row, jnp.asarray(_A_MAP), jnp.asarray(_B_MAP))
    quantized = _make_vq()(param, tree.reshape(T_LEN))
    return (quantized, y)


# speedup vs baseline: 2.0875x; 1.1121x over previous
import dataclasses
import functools

import numpy as np

import jax
import jax.numpy as jnp
from jax import lax
from jax.experimental import pallas as pl
from jax.experimental.pallas import tpu as pltpu
from jax.experimental.pallas import tpu_sc as plsc

N = 524288
K = 512
NC = 2
NS = 16
NW = NC * NS
L = 16
CHUNK = N // NW
T_LEN = 2 * K
REG_LEVELS = 3
GATHER_LEVELS = 9 - REG_LEVELS


def _tree_index_maps():
    perm = np.zeros(K - 1, np.int32)
    for j in range(K - 1):
        lvl = (j + 1).bit_length() - 1
        pos = j - ((1 << lvl) - 1)
        span = K >> lvl
        perm[j] = pos * span + span // 2 - 1
    a = np.zeros(T_LEN, np.int32)
    b = np.zeros(T_LEN, np.int32)
    a[: K - 1] = perm
    b[: K - 1] = perm + 1
    a[K - 1 : T_LEN - 1] = b[K - 1 : T_LEN - 1] = np.arange(K)
    return a.reshape(1, T_LEN), b.reshape(1, T_LEN)


_A_MAP, _B_MAP = _tree_index_maps()


def _prep_body(cb_col_ref, cb_row_ref, a_ref, b_ref, t_ref):
    ci = cb_col_ref[...]
    cj = cb_row_ref[...]
    ii = lax.broadcasted_iota(jnp.int32, (K, K), 0)
    jj = lax.broadcasted_iota(jnp.int32, (K, K), 1)
    gt = (ci > cj) | ((ci == cj) & (ii > jj))
    rank = jnp.sum(gt.astype(jnp.int32), axis=1, keepdims=True)
    sel = (rank == a_ref[...]).astype(jnp.float32) + (
        rank == b_ref[...]
    ).astype(jnp.float32)
    t_ref[...] = 0.5 * jnp.sum(sel * ci, axis=0, keepdims=True)


_prep = pl.pallas_call(
    _prep_body,
    out_shape=jax.ShapeDtypeStruct((1, T_LEN), jnp.float32),
)


def _vq_body(param_hbm, t_hbm, out_hbm, t_v, x_v, o_v):
    wid = lax.axis_index("s") * NC + lax.axis_index("c")
    base = wid * CHUNK
    pltpu.sync_copy(t_hbm, t_v)
    pltpu.sync_copy(param_hbm.at[pl.ds(base, CHUNK)], x_v)

    sp = t_v[pl.ds(0, L)]
    splits = [jnp.full((L,), sp[t]) for t in range((1 << REG_LEVELS) - 1)]

    @plsc.parallel_loop(0, CHUNK, step=L, unroll=4)
    def _(i):
        x = x_v[pl.ds(i, L)]
        j = jnp.full((L,), (1 << REG_LEVELS) - 1, jnp.int32)
        for s in splits:
            j = jnp.where(s <= x, j + 1, j)
        for _lvl in range(GATHER_LEVELS):
            m = plsc.load_gather(t_v, [j])
            j = j + j + jnp.where(m <= x, 2, 1)
        o_v[pl.ds(i, L)] = plsc.load_gather(t_v, [j])

    pltpu.sync_copy(o_v, out_hbm.at[pl.ds(base, CHUNK)])


@functools.cache
def _make_vq():
    cp = pltpu.CompilerParams()
    if "needs_layout_passes" in pltpu.CompilerParams.__dataclass_fields__:
        cp = dataclasses.replace(cp, needs_layout_passes=False)
    return pl.kernel(
        _vq_body,
        compiler_params=cp,
        out_type=jax.ShapeDtypeStruct((N,), jnp.float32),
        mesh=plsc.VectorSubcoreMesh(core_axis_name="c", subcore_axis_name="s"),
        scratch_types=[
            pltpu.VMEM((T_LEN,), jnp.float32),
            pltpu.VMEM((CHUNK,), jnp.float32),
            pltpu.VMEM((CHUNK,), jnp.float32),
        ],
    )


@jax.jit
def kernel(param, y, codebook):
    cb_col = codebook.reshape(K, 1)
    cb_row = codebook.reshape(1, K)
    tree = _prep(cb_col, cb_row, jnp.asarray(_A_MAP), jnp.asarray(_B_MAP))
    quantized = _make_vq()(param, tree.reshape(T_LEN))
    return (quantized, y)


# speedup vs baseline: 2.2087x; 1.0581x over previous
import dataclasses
import functools

import numpy as np

import jax
import jax.numpy as jnp
from jax import lax
from jax.experimental import pallas as pl
from jax.experimental.pallas import tpu as pltpu
from jax.experimental.pallas import tpu_sc as plsc

N = 524288
K = 512
NC = 2
NS = 16
NW = NC * NS
L = 16
CHUNK = N // NW
T_LEN = 2 * K
REG_LEVELS = 2
GATHER_LEVELS = 9 - REG_LEVELS


def _tree_index_maps():
    perm = np.zeros(K - 1, np.int32)
    for j in range(K - 1):
        lvl = (j + 1).bit_length() - 1
        pos = j - ((1 << lvl) - 1)
        span = K >> lvl
        perm[j] = pos * span + span // 2 - 1
    a = np.zeros(T_LEN, np.int32)
    b = np.zeros(T_LEN, np.int32)
    a[: K - 1] = perm
    b[: K - 1] = perm + 1
    a[K - 1 : T_LEN - 1] = b[K - 1 : T_LEN - 1] = np.arange(K)
    return a.reshape(1, T_LEN), b.reshape(1, T_LEN)


_A_MAP, _B_MAP = _tree_index_maps()


def _prep_body(cb_col_ref, cb_row_ref, a_ref, b_ref, t_ref):
    ci = cb_col_ref[...]
    cj = cb_row_ref[...]
    ii = lax.broadcasted_iota(jnp.int32, (K, K), 0)
    jj = lax.broadcasted_iota(jnp.int32, (K, K), 1)
    gt = (ci > cj) | ((ci == cj) & (ii > jj))
    rank = jnp.sum(gt.astype(jnp.int32), axis=1, keepdims=True)
    sel = (rank == a_ref[...]).astype(jnp.float32) + (
        rank == b_ref[...]
    ).astype(jnp.float32)
    t_ref[...] = 0.5 * jnp.sum(sel * ci, axis=0, keepdims=True)


_prep = pl.pallas_call(
    _prep_body,
    out_shape=jax.ShapeDtypeStruct((1, T_LEN), jnp.float32),
)


def _vq_body(param_hbm, t_hbm, out_hbm, t_v, x_v, o_v):
    wid = lax.axis_index("s") * NC + lax.axis_index("c")
    base = wid * CHUNK
    pltpu.sync_copy(t_hbm, t_v)
    pltpu.sync_copy(param_hbm.at[pl.ds(base, CHUNK)], x_v)

    sp = t_v[pl.ds(0, L)]
    splits = [jnp.full((L,), sp[t]) for t in range((1 << REG_LEVELS) - 1)]

    @plsc.parallel_loop(0, CHUNK, step=L, unroll=4)
    def _(i):
        x = x_v[pl.ds(i, L)]
        j = jnp.full((L,), (1 << REG_LEVELS) - 1, jnp.int32)
        for s in splits:
            j = jnp.where(s <= x, j + 1, j)
        for _lvl in range(GATHER_LEVELS):
            m = plsc.load_gather(t_v, [j])
            j = j + j + jnp.where(m <= x, 2, 1)
        o_v[pl.ds(i, L)] = plsc.load_gather(t_v, [j])

    pltpu.sync_copy(o_v, out_hbm.at[pl.ds(base, CHUNK)])


@functools.cache
def _make_vq():
    cp = pltpu.CompilerParams()
    if "needs_layout_passes" in pltpu.CompilerParams.__dataclass_fields__:
        cp = dataclasses.replace(cp, needs_layout_passes=False)
    return pl.kernel(
        _vq_body,
        compiler_params=cp,
        out_type=jax.ShapeDtypeStruct((N,), jnp.float32),
        mesh=plsc.VectorSubcoreMesh(core_axis_name="c", subcore_axis_name="s"),
        scratch_types=[
            pltpu.VMEM((T_LEN,), jnp.float32),
            pltpu.VMEM((CHUNK,), jnp.float32),
            pltpu.VMEM((CHUNK,), jnp.float32),
        ],
    )


@jax.jit
def kernel(param, y, codebook):
    cb_col = codebook.reshape(K, 1)
    cb_row = codebook.reshape(1, K)
    tree = _prep(cb_col, cb_row, jnp.asarray(_A_MAP), jnp.asarray(_B_MAP))
    quantized = _make_vq()(param, tree.reshape(T_LEN))
    return (quantized, y)
